# baseline (speedup 1.0000x reference)
import functools

import jax
import jax.numpy as jnp
from jax.experimental import pallas as pl
from jax.experimental.pallas import tpu as pltpu

B = 8
NQ = 6
NB = 1024
D = 512
T = 512
F32 = jnp.float32
BF16 = jnp.bfloat16


def _split(x):
    xh = x.astype(BF16)
    xl = (x - xh.astype(F32)).astype(BF16)
    return xh, xl


def _dot1(a, b):
    return jax.lax.dot(a, b, preferred_element_type=F32)


def _dot2(wbf, xh, xl):
    return _dot1(wbf, xh) + _dot1(wbf, xl)


def _zero(scr):
    scr[...] = jnp.zeros(scr.shape, F32)


def _wprep(w):
    return jnp.transpose(w, (2, 0, 1)).astype(BF16)


def _pad_cols(x2, L, d):
    C = x2.shape[0]
    x3 = x2.reshape(C, B, L)
    x3 = jnp.pad(x3, ((0, 0), (0, 0), (d, d)))
    return x3.reshape(C, B * (L + 2 * d))


def _deint(x2, L):
    C = x2.shape[0]
    Lh = L // 2
    xp3 = jnp.pad(x2.reshape(C, B, L), ((0, 0), (0, 0), (1, 1)))
    xd = xp3.reshape(C, B, Lh + 1, 2)
    return (xd[..., 0].reshape(C, B * (Lh + 1)),
            xd[..., 1].reshape(C, B * (Lh + 1)))


def _copy_in(scr, src, L, off):
    Lp = scr.shape[1] // B
    v = src[...]
    for bb in range(B):
        scr[:, bb * Lp + off:bb * Lp + off + L] = v[:, bb * L:(bb + 1) * L]


def _store(dst, val, L, Lp_s, off):
    Lp_d = dst.shape[1] // B
    for bb in range(B):
        dst[:, bb * Lp_d + off:bb * Lp_d + off + L] = (
            val[:, bb * Lp_s:bb * Lp_s + L])


def _conv_from(scr, w_ref, b_ref, K, d, relu_in=False, relu_out=False):
    X = scr[...]
    if relu_in:
        X = jnp.maximum(X, 0.0)
    Xh, Xl = _split(X)
    Lp = X.shape[1] // B
    W = B * Lp - (K - 1) * d
    acc = None
    for k in range(K):
        p = _dot2(w_ref[k], Xh[:, k * d:k * d + W], Xl[:, k * d:k * d + W])
        acc = p if acc is None else acc + p
    acc = acc + b_ref[...]
    if relu_out:
        acc = jnp.maximum(acc, 0.0)
    return acc, Lp


def _res_layer(src, dst, L, dd, off_d, w1, b1, w2, b2):
    X = src[...]
    Lp = X.shape[1] // B
    Xr = jnp.maximum(X, 0.0)
    Xh, Xl = _split(Xr)
    W = B * Lp - 2 * dd
    acc = None
    for k in range(3):
        p = _dot2(w1[k], Xh[:, k * dd:k * dd + W], Xl[:, k * dd:k * dd + W])
        acc = p if acc is None else acc + p
    h = jnp.maximum(acc + b1[...], 0.0)
    hh, hl = _split(h)
    t = _dot2(w2[...], hh, hl) + b2[...]
    Lp_d = dst.shape[1] // B
    for bb in range(B):
        dst[:, bb * Lp_d + off_d:bb * Lp_d + off_d + L] = (
            X[:, bb * Lp + dd:bb * Lp + dd + L]
            + t[:, bb * Lp:bb * Lp + L])


def _down_val(xe_ref, xo_ref, w_ref, b_ref, Lh):
    Xeh, Xel = _split(xe_ref[...])
    Xoh, Xol = _split(xo_ref[...])
    W = B * (Lh + 1) - 1
    acc = (_dot2(w_ref[0], Xeh[:, 0:W], Xel[:, 0:W])
           + _dot2(w_ref[1], Xoh[:, 0:W], Xol[:, 0:W])
           + _dot2(w_ref[2], Xeh[:, 1:1 + W], Xel[:, 1:1 + W])
           + _dot2(w_ref[3], Xoh[:, 1:1 + W], Xol[:, 1:1 + W])
           + b_ref[...])
    return acc


def _up_store(scr, w_ref, b_ref, oe_ref, oo_ref, L):
    X = scr[...]
    Xh, Xl = _split(X)
    Lp = X.shape[1] // B
    W = B * Lp - 2
    s0 = (slice(None), slice(0, W))
    s1 = (slice(None), slice(1, 1 + W))
    s2 = (slice(None), slice(2, 2 + W))
    ev = (_dot2(w_ref[0], Xh[s0], Xl[s0]) + _dot2(w_ref[1], Xh[s1], Xl[s1])
          + _dot2(w_ref[2], Xh[s1], Xl[s1]) + b_ref[...])
    od = (_dot2(w_ref[0], Xh[s1], Xl[s1]) + _dot2(w_ref[1], Xh[s1], Xl[s1])
          + _dot2(w_ref[2], Xh[s2], Xl[s2]) + b_ref[...])
    for bb in range(B):
        oe_ref[:, bb * L:(bb + 1) * L] = ev[:, bb * Lp:bb * Lp + L]
        oo_ref[:, bb * L:(bb + 1) * L] = od[:, bb * Lp:bb * Lp + L]


def _in_body(x_ref, w_ref, b_ref, o_ref):
    acc, Lp = _conv_from(x_ref, w_ref, b_ref, K=3, d=1, relu_out=True)
    _store(o_ref, acc, 512, Lp, 0)


def _enc_stage_body(xe, xo, wd, bd, w1a, b1a, w2a, b2a, w1b, b1b, w2b, b2b,
                    w1c, b1c, w2c, b2c, o_ref, s1, s3, s9, *, L, conv_out):
    _zero(s1)
    _zero(s3)
    _zero(s9)
    dv = _down_val(xe, xo, wd, bd, L)
    _store(s1, dv, L, L + 1, 1)
    _res_layer(s1, s3, L, 1, 3, w1a, b1a, w2a, b2a)
    _res_layer(s3, s9, L, 3, 9, w1b, b1b, w2b, b2b)
    if conv_out is None:
        _res_layer(s9, o_ref, L, 9, 0, w1c, b1c, w2c, b2c)
    else:
        wo, bo = conv_out
        _res_layer(s9, s1, L, 9, 1, w1c, b1c, w2c, b2c)
        acc, Lp = _conv_from(s1, wo, bo, K=3, d=1)
        _store(o_ref, acc, L, Lp, 0)


def _dec_stage_body(g_ref, w1a, b1a, w2a, b2a, w1b, b1b, w2b, b2b,
                    w1c, b1c, w2c, b2c, wu, bu, oe, oo, s9, s3, s1, s1b,
                    *, L, conv_in):
    _zero(s9)
    _zero(s3)
    _zero(s1)
    _zero(s1b)
    if conv_in is None:
        _copy_in(s9, g_ref, L, 9)
    else:
        wi, bi = conv_in
        _copy_in(s1b, g_ref, L, 1)
        acc, Lp = _conv_from(s1b, wi, bi, K=3, d=1, relu_out=True)
        _store(s9, acc, L, Lp, 9)
    _res_layer(s9, s3, L, 9, 3, w1a, b1a, w2a, b2a)
    _res_layer(s3, s1, L, 3, 1, w1b, b1b, w2b, b2b)
    _res_layer(s1, s1b, L, 1, 1, w1c, b1c, w2c, b2c)
    _up_store(s1b, wu, bu, oe, oo, L)


def _out_body(g_ref, w1, b1, w2, b2, o_ref, sa, sb):
    _zero(sa)
    _zero(sb)
    _copy_in(sa, g_ref, 512, 1)
    acc, Lp = _conv_from(sa, w1, b1, K=3, d=1, relu_out=True)
    _store(sb, acc, 512, Lp, 1)
    acc2, Lp2 = _conv_from(sb, w2, b2, K=3, d=1)
    _store(o_ref, acc2, 512, Lp2, 0)


def _rvq_body(z_ref, cb_ref, cbt_ref, xq_ref, cl_ref, pp_ref):
    r = z_ref[...]
    qsum = jnp.zeros_like(r)
    cl = jnp.zeros((1, 1), F32)
    pp = jnp.zeros((1, 1), F32)
    iota = jax.lax.broadcasted_iota(jnp.int32, (T, NB), 1)
    for l in range(NQ):
        cbt = cbt_ref[l]
        cb = cb_ref[l]
        cbt_bf = cbt.astype(BF16)
        cn = jnp.sum(cbt * cbt, axis=0, keepdims=True)
        rh, rl = _split(r)
        sc = _dot1(rh, cbt_bf) + _dot1(rl, cbt_bf)
        s = cn - 2.0 * sc
        m = jnp.min(s, axis=1, keepdims=True)
        idx = jnp.min(jnp.where(s == m, iota, NB), axis=1, keepdims=True)
        oh = (iota == idx).astype(F32)
        q = jax.lax.dot(oh, cb, precision=jax.lax.Precision.HIGHEST,
                        preferred_element_type=F32)
        diff = r - q
        cl = cl + jnp.sum(diff * diff, keepdims=True) * (1.0 / (T * D))
        counts = jnp.sum(oh, axis=0, keepdims=True)
        probs = counts * (1.0 / T)
        ent = jnp.sum(probs * jnp.log(probs + 1e-10), keepdims=True)
        pp = pp + jnp.exp(-ent)
        qsum = qsum + q
        r = diff
    xq_ref[...] = qsum
    cl_ref[...] = cl
    pp_ref[...] = pp * (1.0 / NQ)


def _scr(L, d):
    return pltpu.VMEM((512, B * (L + 2 * d)), F32)


def _res_args(rb):
    return (_wprep(rb["c1"]["w"]), rb["c1"]["b"][:, None],
            rb["c2"]["w"][:, :, 0].astype(BF16), rb["c2"]["b"][:, None])


def _enc_stage(h, blk, L, conv_out_p):
    Lh = L // 2
    xe, xo = _deint(h, L)
    args = [xe, xo, _wprep(blk["down"]["w"]), blk["down"]["b"][:, None]]
    for rb in blk["res"]:
        args.extend(_res_args(rb))
    if conv_out_p is None:
        return pl.pallas_call(
            functools.partial(_enc_stage_body, L=Lh, conv_out=None),
            out_shape=jax.ShapeDtypeStruct((512, B * Lh), F32),
            scratch_shapes=[_scr(Lh, 1), _scr(Lh, 3), _scr(Lh, 9)],
        )(*args)

    wo = _wprep(conv_out_p["w"])
    bo = conv_out_p["b"][:, None]

    def body2(xe, xo, wd, bd, w1a, b1a, w2a, b2a, w1b, b1b, w2b, b2b,
              w1c, b1c, w2c, b2c, wo_r, bo_r, o_ref, s1, s3, s9):
        _enc_stage_body(xe, xo, wd, bd, w1a, b1a, w2a, b2a, w1b, b1b, w2b,
                        b2b, w1c, b1c, w2c, b2c, o_ref, s1, s3, s9,
                        L=Lh, conv_out=(wo_r, bo_r))

    return pl.pallas_call(
        body2,
        out_shape=jax.ShapeDtypeStruct((512, B * Lh), F32),
        scratch_shapes=[_scr(Lh, 1), _scr(Lh, 3), _scr(Lh, 9)],
    )(*args, wo, bo)


def _dec_stage(g, blk, L, conv_in_p):
    args = [g]
    for rb in blk["res"]:
        args.extend(_res_args(rb))
    wu = _wprep(blk["conv"]["w"])
    bu = blk["conv"]["b"][:, None]
    sh = jax.ShapeDtypeStruct((512, B * L), F32)
    scr = [_scr(L, 9), _scr(L, 3), _scr(L, 1), _scr(L, 1)]
    if conv_in_p is None:
        ev, od = pl.pallas_call(
            functools.partial(_dec_stage_body, L=L, conv_in=None),
            out_shape=(sh, sh),
            scratch_shapes=scr,
        )(*args, wu, bu)
    else:
        wi = _wprep(conv_in_p["w"])
        bi = conv_in_p["b"][:, None]

        def body2(g_ref, wi_r, bi_r, w1a, b1a, w2a, b2a, w1b, b1b, w2b, b2b,
                  w1c, b1c, w2c, b2c, wu_r, bu_r, oe, oo, s9, s3, s1, s1b):
            _dec_stage_body(g_ref, w1a, b1a, w2a, b2a, w1b, b1b, w2b, b2b,
                            w1c, b1c, w2c, b2c, wu_r, bu_r, oe, oo,
                            s9, s3, s1, s1b, L=L, conv_in=(wi_r, bi_r))

        ev, od = pl.pallas_call(
            body2,
            out_shape=(sh, sh),
            scratch_shapes=scr,
        )(args[0], wi, bi, *args[1:], wu, bu)
    y = jnp.stack([ev.reshape(512, B, L), od.reshape(512, B, L)], axis=3)
    return y.reshape(512, B * 2 * L)


def _rvq(z, codebooks):
    cbt = jnp.transpose(codebooks, (0, 2, 1))
    return pl.pallas_call(
        _rvq_body,
        out_shape=(
            jax.ShapeDtypeStruct((T, D), F32),
            jax.ShapeDtypeStruct((1, 1), F32),
            jax.ShapeDtypeStruct((1, 1), F32),
        ),
    )(z, codebooks, cbt)


def kernel(x, params):
    enc = params["enc"]
    dec = params["dec"]

    x2 = jnp.transpose(x, (2, 0, 1)).reshape(263, B * 512)
    x2 = jnp.pad(x2, ((0, 1), (0, 0)))
    xp = _pad_cols(x2, 512, 1)
    w_in = jnp.pad(enc["conv_in"]["w"], ((0, 0), (0, 1), (0, 0)))
    h = pl.pallas_call(
        _in_body,
        out_shape=jax.ShapeDtypeStruct((512, B * 512), F32),
    )(xp, _wprep(w_in), enc["conv_in"]["b"][:, None])

    L = 512
    for i, blk in enumerate(enc["downs"]):
        co = enc["conv_out"] if i == 2 else None
        h = _enc_stage(h, blk, L, co)
        L //= 2

    z = h.T
    xq, cl, pp = _rvq(z, params["codebooks"])
    commit_loss = cl[0, 0]
    perplexity = pp[0, 0]

    g = xq.T
    L = 64
    for i, blk in enumerate(dec["ups"]):
        ci = dec["conv_in"] if i == 0 else None
        g = _dec_stage(g, blk, L, ci)
        L *= 2

    w2 = jnp.pad(dec["conv_out2"]["w"], ((0, 1), (0, 0), (0, 0)))
    b2 = jnp.pad(dec["conv_out2"]["b"], ((0, 1),))
    g = pl.pallas_call(
        _out_body,
        out_shape=jax.ShapeDtypeStruct((264, B * 512), F32),
        scratch_shapes=[_scr(512, 1), _scr(512, 1)],
    )(g, _wprep(dec["conv_out1"]["w"]), dec["conv_out1"]["b"][:, None],
      _wprep(w2), b2[:, None])

    x_out = jnp.transpose(g.reshape(264, B, 512), (1, 0, 2))[:, :263, :]
    return (x_out, commit_loss, perplexity)
